# baseline (speedup 1.0000x reference)
import functools

import jax
import jax.numpy as jnp
from jax import lax
from jax.experimental import pallas as pl
from jax.experimental.pallas import tpu as pltpu
from jax.experimental.pallas import tpu_sc as plsc

N = 10000
E = 320000
D = 128

NC = 2
NS = 16
NW = NC * NS

NPAD = 10112
ROWS_PER_TILE = NPAD // NS
EPW = E // NW
CHUNK = 128
NBUF = 3
EPW_P = 10368
NCHUNK = EPW_P // CHUNK
GROUPS = NCHUNK // NBUF


def _sc_segment_sum_body(x_hbm, idx_hbm, zeros_hbm, out_hbm,
                         sseg, rows, idx, gsem, ssem, isem):
    cid = lax.axis_index("c")
    sid = lax.axis_index("s")
    wid = sid * NC + cid

    rbase = sid * ROWS_PER_TILE
    pltpu.sync_copy(zeros_hbm, sseg.at[pl.ds(rbase, ROWS_PER_TILE)])

    def idx_load(c, b):
        pltpu.async_copy(idx_hbm.at[wid, c], idx.at[b], isem.at[b])

    def idx_wait(b):
        pltpu.make_async_copy(idx_hbm.at[wid, 0], idx.at[b],
                              isem.at[b]).wait()

    def gather(b):
        pltpu.async_copy(x_hbm.at[idx.at[b, 0]], rows.at[b], gsem.at[b])

    def gather_wait(b):
        pltpu.make_async_copy(x_hbm.at[idx.at[b, 0]], rows.at[b],
                              gsem.at[b]).wait()

    def scatter(b):
        pltpu.async_copy(rows.at[b], sseg.at[idx.at[b, 1]], ssem.at[b],
                         add=True)

    def scatter_wait(b):
        pltpu.make_async_copy(rows.at[b], sseg.at[idx.at[b, 1]],
                              ssem.at[b]).wait()

    idx_load(0, 0)
    idx_load(1, 1)
    idx_wait(0)
    gather(0)
    plsc.subcore_barrier()

    def group(g, carry):
        for b in range(NBUF):
            bn = (b + 1) % NBUF
            bnn = (b + 2) % NBUF
            c = g * NBUF + b
            gather_wait(b)
            scatter(b)

            @pl.when(c + 1 < NCHUNK)
            def _():
                idx_wait(bn)
                gather(bn)

            @pl.when(c >= 1)
            def _():
                scatter_wait(bnn)

            @pl.when(c + 2 < NCHUNK)
            def _():
                idx_load(c + 2, bnn)

        return carry

    lax.fori_loop(0, GROUPS, group, 0)
    scatter_wait((NCHUNK - 1) % NBUF)
    plsc.subcore_barrier()

    sl = pl.ds(rbase, ROWS_PER_TILE)
    pltpu.sync_copy(sseg.at[sl], out_hbm.at[cid, sl])


@functools.partial(jax.jit, static_argnames=())
def _sc_segment_sum(x, idx, zeros):
    mesh = plsc.VectorSubcoreMesh(core_axis_name="c", subcore_axis_name="s")
    return pl.kernel(
        _sc_segment_sum_body,
        out_type=jax.ShapeDtypeStruct((NC, NPAD, D), jnp.float32),
        mesh=mesh,
        scratch_types=[
            pltpu.VMEM_SHARED((NPAD, D), jnp.float32),
            pltpu.VMEM((NBUF, CHUNK, D), jnp.float32),
            pltpu.VMEM((NBUF, 2, CHUNK), jnp.int32),
            pltpu.SemaphoreType.DMA((NBUF,)),
            pltpu.SemaphoreType.DMA((NBUF,)),
            pltpu.SemaphoreType.DMA((NBUF,)),
        ],
    )(x, idx, zeros)


ROW_BLK = 400
NBLK = N // ROW_BLK


def _tc_mlp_body(eps_ref, x_ref, s_ref, wa_ref, ba_ref, wb_ref, bb_ref, o_ref):
    z = (1.0 + eps_ref[0]) * x_ref[...] + s_ref[0] + s_ref[1]
    h = jnp.dot(z, wa_ref[...], preferred_element_type=jnp.float32) + ba_ref[...]
    h = jnp.maximum(h, 0.0)
    o_ref[...] = jnp.dot(h, wb_ref[...], preferred_element_type=jnp.float32) + bb_ref[...]


def _tc_mlp(eps, x, spair, wa, ba, wb, bb):
    return pl.pallas_call(
        _tc_mlp_body,
        grid=(NBLK,),
        in_specs=[
            pl.BlockSpec(memory_space=pltpu.SMEM),
            pl.BlockSpec((ROW_BLK, D), lambda i: (i, 0)),
            pl.BlockSpec((NC, ROW_BLK, D), lambda i: (0, i, 0)),
            pl.BlockSpec((D, D), lambda i: (0, 0)),
            pl.BlockSpec((1, D), lambda i: (0, 0)),
            pl.BlockSpec((D, D), lambda i: (0, 0)),
            pl.BlockSpec((1, D), lambda i: (0, 0)),
        ],
        out_specs=pl.BlockSpec((ROW_BLK, D), lambda i: (i, 0)),
        out_shape=jax.ShapeDtypeStruct((N, D), jnp.float32),
    )(eps, x, spair, wa, ba, wb, bb)


def kernel(X, edge_index, eps1, W1a, b1a, W1b, b1b, eps2, W2a, b2a, W2b, b2b):
    src = edge_index[0].reshape(NW, EPW)
    dst = edge_index[1].reshape(NW, EPW)
    src = jnp.pad(src, ((0, 0), (0, EPW_P - EPW)))
    dst = jnp.pad(dst, ((0, 0), (0, EPW_P - EPW)),
                  constant_values=NPAD - 1)
    idx = jnp.stack([src.reshape(NW, NCHUNK, CHUNK),
                     dst.reshape(NW, NCHUNK, CHUNK)], axis=2)
    zeros = jnp.zeros((ROWS_PER_TILE, D), jnp.float32)
    b1a2 = b1a.reshape(1, D)
    b1b2 = b1b.reshape(1, D)
    b2a2 = b2a.reshape(1, D)
    b2b2 = b2b.reshape(1, D)

    s1 = _sc_segment_sum(X, idx, zeros)
    x1 = _tc_mlp(eps1, X, s1, W1a, b1a2, W1b, b1b2)
    s2 = _sc_segment_sum(x1, idx, zeros)
    x2 = _tc_mlp(eps2, x1, s2, W2a, b2a2, W2b, b2b2)
    return x2


# speedup vs baseline: 1.0009x; 1.0009x over previous
import functools

import jax
import jax.numpy as jnp
from jax import lax
from jax.experimental import pallas as pl
from jax.experimental.pallas import tpu as pltpu
from jax.experimental.pallas import tpu_sc as plsc

N = 10000
E = 320000
D = 128

NC = 2
NS = 16
NW = NC * NS

NPAD = 10112
ROWS_PER_TILE = NPAD // NS
EPW = E // NW
CHUNK = 128
NBUF = 3
EPW_P = 10368
NCHUNK = EPW_P // CHUNK
GROUPS = NCHUNK // NBUF


def _sc_segment_sum_body(x_hbm, idx_hbm, zeros_hbm, out_hbm,
                         sseg, rows, idx, gsem, ssem, isem):
    cid = lax.axis_index("c")
    sid = lax.axis_index("s")
    wid = sid * NC + cid

    rbase = sid * ROWS_PER_TILE
    pltpu.sync_copy(zeros_hbm, rows.at[0])
    for j in range(4):
        pltpu.sync_copy(rows.at[0],
                        sseg.at[pl.ds(rbase + j * CHUNK, CHUNK)])
    pltpu.sync_copy(rows.at[0, pl.ds(0, ROWS_PER_TILE - 4 * CHUNK)],
                    sseg.at[pl.ds(rbase + 4 * CHUNK, ROWS_PER_TILE - 4 * CHUNK)])

    def idx_load(c, b):
        pltpu.async_copy(idx_hbm.at[wid, c], idx.at[b], isem.at[b])

    def idx_wait(b):
        pltpu.make_async_copy(idx_hbm.at[wid, 0], idx.at[b],
                              isem.at[b]).wait()

    def gather(b):
        pltpu.async_copy(x_hbm.at[idx.at[b, 0]], rows.at[b], gsem.at[b])

    def gather_wait(b):
        pltpu.make_async_copy(x_hbm.at[idx.at[b, 0]], rows.at[b],
                              gsem.at[b]).wait()

    def scatter(b):
        pltpu.async_copy(rows.at[b], sseg.at[idx.at[b, 1]], ssem.at[b],
                         add=True)

    def scatter_wait(b):
        pltpu.make_async_copy(rows.at[b], sseg.at[idx.at[b, 1]],
                              ssem.at[b]).wait()

    idx_load(0, 0)
    idx_load(1, 1)
    idx_wait(0)
    gather(0)
    plsc.subcore_barrier()

    def group(g, carry):
        for b in range(NBUF):
            bn = (b + 1) % NBUF
            bnn = (b + 2) % NBUF
            c = g * NBUF + b
            gather_wait(b)
            scatter(b)

            @pl.when(c + 1 < NCHUNK)
            def _():
                idx_wait(bn)
                gather(bn)

            @pl.when(c >= 1)
            def _():
                scatter_wait(bnn)

            @pl.when(c + 2 < NCHUNK)
            def _():
                idx_load(c + 2, bnn)

        return carry

    lax.fori_loop(0, GROUPS, group, 0)
    scatter_wait((NCHUNK - 1) % NBUF)
    plsc.subcore_barrier()

    for j in range(4):
        sl = pl.ds(rbase + j * CHUNK, CHUNK)
        pltpu.sync_copy(sseg.at[sl], rows.at[0])
        pltpu.sync_copy(rows.at[0], out_hbm.at[cid, sl])
    tail = ROWS_PER_TILE - 4 * CHUNK
    sl = pl.ds(rbase + 4 * CHUNK, tail)
    pltpu.sync_copy(sseg.at[sl], rows.at[0, pl.ds(0, tail)])
    pltpu.sync_copy(rows.at[0, pl.ds(0, tail)], out_hbm.at[cid, sl])


@functools.partial(jax.jit, static_argnames=())
def _sc_segment_sum(x, idx, zeros):
    mesh = plsc.VectorSubcoreMesh(core_axis_name="c", subcore_axis_name="s")
    return pl.kernel(
        _sc_segment_sum_body,
        out_type=jax.ShapeDtypeStruct((NC, NPAD, D), jnp.float32),
        mesh=mesh,
        scratch_types=[
            pltpu.VMEM_SHARED((NPAD, D), jnp.float32),
            pltpu.VMEM((NBUF, CHUNK, D), jnp.float32),
            pltpu.VMEM((NBUF, 2, CHUNK), jnp.int32),
            pltpu.SemaphoreType.DMA((NBUF,)),
            pltpu.SemaphoreType.DMA((NBUF,)),
            pltpu.SemaphoreType.DMA((NBUF,)),
        ],
    )(x, idx, zeros)


ROW_BLK = 400
NBLK = N // ROW_BLK


def _tc_mlp_body(eps_ref, x_ref, s_ref, wa_ref, ba_ref, wb_ref, bb_ref, o_ref):
    z = (1.0 + eps_ref[0]) * x_ref[...] + s_ref[0] + s_ref[1]
    h = jnp.dot(z, wa_ref[...], preferred_element_type=jnp.float32) + ba_ref[...]
    h = jnp.maximum(h, 0.0)
    o_ref[...] = jnp.dot(h, wb_ref[...], preferred_element_type=jnp.float32) + bb_ref[...]


def _tc_mlp(eps, x, spair, wa, ba, wb, bb):
    return pl.pallas_call(
        _tc_mlp_body,
        grid=(NBLK,),
        in_specs=[
            pl.BlockSpec(memory_space=pltpu.SMEM),
            pl.BlockSpec((ROW_BLK, D), lambda i: (i, 0)),
            pl.BlockSpec((NC, ROW_BLK, D), lambda i: (0, i, 0)),
            pl.BlockSpec((D, D), lambda i: (0, 0)),
            pl.BlockSpec((1, D), lambda i: (0, 0)),
            pl.BlockSpec((D, D), lambda i: (0, 0)),
            pl.BlockSpec((1, D), lambda i: (0, 0)),
        ],
        out_specs=pl.BlockSpec((ROW_BLK, D), lambda i: (i, 0)),
        out_shape=jax.ShapeDtypeStruct((N, D), jnp.float32),
    )(eps, x, spair, wa, ba, wb, bb)


def kernel(X, edge_index, eps1, W1a, b1a, W1b, b1b, eps2, W2a, b2a, W2b, b2b):
    src = edge_index[0].reshape(NW, EPW)
    dst = edge_index[1].reshape(NW, EPW)
    src = jnp.pad(src, ((0, 0), (0, EPW_P - EPW)))
    dst = jnp.pad(dst, ((0, 0), (0, EPW_P - EPW)),
                  constant_values=NPAD - 1)
    idx = jnp.stack([src.reshape(NW, NCHUNK, CHUNK),
                     dst.reshape(NW, NCHUNK, CHUNK)], axis=2)
    zeros = jnp.zeros((CHUNK, D), jnp.float32)
    b1a2 = b1a.reshape(1, D)
    b1b2 = b1b.reshape(1, D)
    b2a2 = b2a.reshape(1, D)
    b2b2 = b2b.reshape(1, D)

    s1 = _sc_segment_sum(X, idx, zeros)
    x1 = _tc_mlp(eps1, X, s1, W1a, b1a2, W1b, b1b2)
    s2 = _sc_segment_sum(x1, idx, zeros)
    x2 = _tc_mlp(eps2, x1, s2, W2a, b2a2, W2b, b2b2)
    return x2


# speedup vs baseline: 2.1341x; 2.1321x over previous
import functools

import jax
import jax.numpy as jnp
from jax import lax
from jax.experimental import pallas as pl
from jax.experimental.pallas import tpu as pltpu
from jax.experimental.pallas import tpu_sc as plsc

N = 10000
E = 320000
D = 128

NC = 2
NS = 16
NW = NC * NS

NPAD = 10112
ROWS_PER_TILE = NPAD // NS
EPW = E // NW
CHUNK = 80
NBUF = 3
EPW_P = 10080
NCHUNK = EPW_P // CHUNK
GROUPS = NCHUNK // NBUF
ZFULL = ROWS_PER_TILE // CHUNK
ZTAIL = ROWS_PER_TILE % CHUNK


def _sc_segment_sum_body(x_hbm, idx_hbm, zeros_hbm, out_hbm,
                         sseg, rows, idx, gsem, ssem, isem):
    cid = lax.axis_index("c")
    sid = lax.axis_index("s")
    wid = sid * NC + cid

    rbase = sid * ROWS_PER_TILE
    pltpu.sync_copy(zeros_hbm, rows.at[0])
    for j in range(ZFULL):
        pltpu.sync_copy(rows.at[0],
                        sseg.at[pl.ds(rbase + j * CHUNK, CHUNK)])
    if ZTAIL:
        pltpu.sync_copy(rows.at[0, pl.ds(0, ZTAIL)],
                        sseg.at[pl.ds(rbase + ZFULL * CHUNK, ZTAIL)])

    def idx_load(c, b):
        pltpu.async_copy(idx_hbm.at[wid, c], idx.at[b], isem.at[b])

    def idx_wait(b):
        pltpu.make_async_copy(idx_hbm.at[wid, 0], idx.at[b],
                              isem.at[b]).wait()

    def gather(b):
        pltpu.async_copy(x_hbm.at[idx.at[b, 0]], rows.at[b], gsem.at[b])

    def gather_wait(b):
        pltpu.make_async_copy(x_hbm.at[idx.at[b, 0]], rows.at[b],
                              gsem.at[b]).wait()

    def scatter(b):
        pltpu.async_copy(rows.at[b], sseg.at[idx.at[b, 1]], ssem.at[b],
                         add=True)

    def scatter_wait(b):
        pltpu.make_async_copy(rows.at[b], sseg.at[idx.at[b, 1]],
                              ssem.at[b]).wait()

    idx_load(0, 0)
    idx_load(1, 1)
    idx_wait(0)
    gather(0)
    plsc.subcore_barrier()

    def group(g, carry):
        for b in range(NBUF):
            bn = (b + 1) % NBUF
            bnn = (b + 2) % NBUF
            c = g * NBUF + b
            gather_wait(b)
            scatter(b)

            @pl.when(c + 1 < NCHUNK)
            def _():
                idx_wait(bn)
                gather(bn)

            @pl.when(c >= 1)
            def _():
                scatter_wait(bnn)

            @pl.when(c + 2 < NCHUNK)
            def _():
                idx_load(c + 2, bnn)

        return carry

    lax.fori_loop(0, GROUPS, group, 0)
    scatter_wait((NCHUNK - 1) % NBUF)
    plsc.subcore_barrier()

    for j in range(ZFULL):
        sl = pl.ds(rbase + j * CHUNK, CHUNK)
        pltpu.sync_copy(sseg.at[sl], rows.at[0])
        pltpu.sync_copy(rows.at[0], out_hbm.at[cid, sl])
    if ZTAIL:
        sl = pl.ds(rbase + ZFULL * CHUNK, ZTAIL)
        pltpu.sync_copy(sseg.at[sl], rows.at[0, pl.ds(0, ZTAIL)])
        pltpu.sync_copy(rows.at[0, pl.ds(0, ZTAIL)], out_hbm.at[cid, sl])


@functools.partial(jax.jit, static_argnames=())
def _sc_segment_sum(x, idx, zeros):
    mesh = plsc.VectorSubcoreMesh(core_axis_name="c", subcore_axis_name="s")
    return pl.kernel(
        _sc_segment_sum_body,
        out_type=jax.ShapeDtypeStruct((NC, NPAD, D), jnp.float32),
        mesh=mesh,
        scratch_types=[
            pltpu.VMEM_SHARED((NPAD, D), jnp.float32),
            pltpu.VMEM((NBUF, CHUNK, D), jnp.float32),
            pltpu.VMEM((NBUF, 2, CHUNK), jnp.int32),
            pltpu.SemaphoreType.DMA((NBUF,)),
            pltpu.SemaphoreType.DMA((NBUF,)),
            pltpu.SemaphoreType.DMA((NBUF,)),
        ],
    )(x, idx, zeros)


ROW_BLK = 400
NBLK = N // ROW_BLK


def _tc_mlp_body(eps_ref, x_ref, s_ref, wa_ref, ba_ref, wb_ref, bb_ref, o_ref):
    z = (1.0 + eps_ref[0]) * x_ref[...] + s_ref[0] + s_ref[1]
    h = jnp.dot(z, wa_ref[...], preferred_element_type=jnp.float32) + ba_ref[...]
    h = jnp.maximum(h, 0.0)
    o_ref[...] = jnp.dot(h, wb_ref[...], preferred_element_type=jnp.float32) + bb_ref[...]


def _tc_mlp(eps, x, spair, wa, ba, wb, bb):
    return pl.pallas_call(
        _tc_mlp_body,
        grid=(NBLK,),
        in_specs=[
            pl.BlockSpec(memory_space=pltpu.SMEM),
            pl.BlockSpec((ROW_BLK, D), lambda i: (i, 0)),
            pl.BlockSpec((NC, ROW_BLK, D), lambda i: (0, i, 0)),
            pl.BlockSpec((D, D), lambda i: (0, 0)),
            pl.BlockSpec((1, D), lambda i: (0, 0)),
            pl.BlockSpec((D, D), lambda i: (0, 0)),
            pl.BlockSpec((1, D), lambda i: (0, 0)),
        ],
        out_specs=pl.BlockSpec((ROW_BLK, D), lambda i: (i, 0)),
        out_shape=jax.ShapeDtypeStruct((N, D), jnp.float32),
    )(eps, x, spair, wa, ba, wb, bb)


def kernel(X, edge_index, eps1, W1a, b1a, W1b, b1b, eps2, W2a, b2a, W2b, b2b):
    src = edge_index[0].reshape(NW, EPW)
    dst = edge_index[1].reshape(NW, EPW)
    src = jnp.pad(src, ((0, 0), (0, EPW_P - EPW)))
    dst = jnp.pad(dst, ((0, 0), (0, EPW_P - EPW)),
                  constant_values=NPAD - 1)
    idx = jnp.stack([src.reshape(NW, NCHUNK, CHUNK),
                     dst.reshape(NW, NCHUNK, CHUNK)], axis=2)
    zeros = jnp.zeros((CHUNK, D), jnp.float32)
    b1a2 = b1a.reshape(1, D)
    b1b2 = b1b.reshape(1, D)
    b2a2 = b2a.reshape(1, D)
    b2b2 = b2b.reshape(1, D)

    s1 = _sc_segment_sum(X, idx, zeros)
    x1 = _tc_mlp(eps1, X, s1, W1a, b1a2, W1b, b1b2)
    s2 = _sc_segment_sum(x1, idx, zeros)
    x2 = _tc_mlp(eps2, x1, s2, W2a, b2a2, W2b, b2b2)
    return x2


# speedup vs baseline: 2.6656x; 1.2491x over previous
import functools

import jax
import jax.numpy as jnp
from jax import lax
from jax.experimental import pallas as pl
from jax.experimental.pallas import tpu as pltpu
from jax.experimental.pallas import tpu_sc as plsc

N = 10000
E = 320000
D = 128

NC = 2
NS = 16
NW = NC * NS

NPAD = 10112
ROWS_PER_TILE = NPAD // NS
EPW = E // NW
CHUNK = 96
NBUF = 3
EPW_P = 10080
NCHUNK = EPW_P // CHUNK
GROUPS = NCHUNK // NBUF
ZFULL = ROWS_PER_TILE // CHUNK
ZTAIL = ROWS_PER_TILE % CHUNK
IDX_BITS = 14
IDX_MASK = (1 << IDX_BITS) - 1


def _sc_segment_sum_body(x_hbm, pidx_hbm, zeros_hbm, out_hbm,
                         sseg, rows, pidx, sidx, didx, gsem, ssem):
    cid = lax.axis_index("c")
    sid = lax.axis_index("s")
    wid = sid * NC + cid

    rbase = sid * ROWS_PER_TILE
    pltpu.sync_copy(zeros_hbm, rows.at[0])
    for j in range(ZFULL):
        pltpu.sync_copy(rows.at[0],
                        sseg.at[pl.ds(rbase + j * CHUNK, CHUNK)])
    if ZTAIL:
        pltpu.sync_copy(rows.at[0, pl.ds(0, ZTAIL)],
                        sseg.at[pl.ds(rbase + ZFULL * CHUNK, ZTAIL)])
    pltpu.sync_copy(pidx_hbm.at[wid], pidx)

    def unpack(c, b):
        for j in range(CHUNK // 16):
            sl = pl.ds(j * 16, 16)
            p = pidx[pl.ds(c * CHUNK + j * 16, 16)]
            sidx[b, sl] = jnp.bitwise_and(p, IDX_MASK)
            didx[b, sl] = lax.shift_right_logical(p, IDX_BITS)

    def gather(c, b):
        pltpu.async_copy(x_hbm.at[sidx.at[b]], rows.at[b], gsem.at[b])

    def gather_wait(b):
        pltpu.make_async_copy(x_hbm.at[sidx.at[b]], rows.at[b],
                              gsem.at[b]).wait()

    def scatter(c, b):
        pltpu.async_copy(rows.at[b], sseg.at[didx.at[b]], ssem.at[b],
                         add=True)

    def scatter_wait(b):
        pltpu.make_async_copy(rows.at[b], sseg.at[didx.at[b]],
                              ssem.at[b]).wait()

    for b in range(NBUF):
        unpack(b, b)
        gather(b, b)
    plsc.subcore_barrier()

    def group(g, carry):
        for b in range(NBUF):
            c = g * NBUF + b
            gather_wait(b)
            scatter(c, b)
            bp = (b - 1) % NBUF
            cp = c - 1

            @pl.when(jnp.logical_and(cp >= 0, cp + NBUF < NCHUNK))
            def _():
                scatter_wait(bp)
                unpack(cp + NBUF, bp)
                gather(cp + NBUF, bp)

        return carry

    lax.fori_loop(0, GROUPS, group, 0)
    for b in range(NBUF):
        scatter_wait(b)
    plsc.subcore_barrier()

    for j in range(ZFULL):
        sl = pl.ds(rbase + j * CHUNK, CHUNK)
        pltpu.sync_copy(sseg.at[sl], rows.at[0])
        pltpu.sync_copy(rows.at[0], out_hbm.at[cid, sl])
    if ZTAIL:
        sl = pl.ds(rbase + ZFULL * CHUNK, ZTAIL)
        pltpu.sync_copy(sseg.at[sl], rows.at[0, pl.ds(0, ZTAIL)])
        pltpu.sync_copy(rows.at[0, pl.ds(0, ZTAIL)], out_hbm.at[cid, sl])


@functools.partial(jax.jit, static_argnames=())
def _sc_segment_sum(x, pidx, zeros):
    mesh = plsc.VectorSubcoreMesh(core_axis_name="c", subcore_axis_name="s")
    return pl.kernel(
        _sc_segment_sum_body,
        out_type=jax.ShapeDtypeStruct((NC, NPAD, D), jnp.float32),
        mesh=mesh,
        scratch_types=[
            pltpu.VMEM_SHARED((NPAD, D), jnp.float32),
            pltpu.VMEM((NBUF, CHUNK, D), jnp.float32),
            pltpu.VMEM((NCHUNK * CHUNK,), jnp.int32),
            pltpu.VMEM((NBUF, CHUNK), jnp.int32),
            pltpu.VMEM((NBUF, CHUNK), jnp.int32),
            pltpu.SemaphoreType.DMA((NBUF,)),
            pltpu.SemaphoreType.DMA((NBUF,)),
        ],
    )(x, pidx, zeros)


ROW_BLK = 400
NBLK = N // ROW_BLK


def _tc_mlp_body(eps_ref, x_ref, s_ref, wa_ref, ba_ref, wb_ref, bb_ref, o_ref):
    z = (1.0 + eps_ref[0]) * x_ref[...] + s_ref[0] + s_ref[1]
    h = jnp.dot(z, wa_ref[...], preferred_element_type=jnp.float32) + ba_ref[...]
    h = jnp.maximum(h, 0.0)
    o_ref[...] = jnp.dot(h, wb_ref[...], preferred_element_type=jnp.float32) + bb_ref[...]


def _tc_mlp(eps, x, spair, wa, ba, wb, bb):
    return pl.pallas_call(
        _tc_mlp_body,
        grid=(NBLK,),
        in_specs=[
            pl.BlockSpec(memory_space=pltpu.SMEM),
            pl.BlockSpec((ROW_BLK, D), lambda i: (i, 0)),
            pl.BlockSpec((NC, ROW_BLK, D), lambda i: (0, i, 0)),
            pl.BlockSpec((D, D), lambda i: (0, 0)),
            pl.BlockSpec((1, D), lambda i: (0, 0)),
            pl.BlockSpec((D, D), lambda i: (0, 0)),
            pl.BlockSpec((1, D), lambda i: (0, 0)),
        ],
        out_specs=pl.BlockSpec((ROW_BLK, D), lambda i: (i, 0)),
        out_shape=jax.ShapeDtypeStruct((N, D), jnp.float32),
    )(eps, x, spair, wa, ba, wb, bb)


def kernel(X, edge_index, eps1, W1a, b1a, W1b, b1b, eps2, W2a, b2a, W2b, b2b):
    src = edge_index[0].reshape(NW, EPW)
    dst = edge_index[1].reshape(NW, EPW)
    src = jnp.pad(src, ((0, 0), (0, EPW_P - EPW)))
    dst = jnp.pad(dst, ((0, 0), (0, EPW_P - EPW)),
                  constant_values=NPAD - 1)
    pidx = (src + (dst << IDX_BITS)).reshape(NW, NCHUNK * CHUNK)
    zeros = jnp.zeros((CHUNK, D), jnp.float32)
    b1a2 = b1a.reshape(1, D)
    b1b2 = b1b.reshape(1, D)
    b2a2 = b2a.reshape(1, D)
    b2b2 = b2b.reshape(1, D)

    s1 = _sc_segment_sum(X, pidx, zeros)
    x1 = _tc_mlp(eps1, X, s1, W1a, b1a2, W1b, b1b2)
    s2 = _sc_segment_sum(x1, pidx, zeros)
    x2 = _tc_mlp(eps2, x1, s2, W2a, b2a2, W2b, b2b2)
    return x2
